# Initial kernel scaffold; baseline (speedup 1.0000x reference)
import functools

import jax
import jax.numpy as jnp
from jax import lax
from jax.experimental import pallas as pl
from jax.experimental.pallas import tpu as pltpu
from jax.experimental.pallas import tpu_sc as plsc

NUM_CORES = 2
NUM_SUBCORES = 16
NUM_WORKERS = NUM_CORES * NUM_SUBCORES

BATCH = 16384
DIM = 64
NLOOKUP = 4

B_PER_W = BATCH // NUM_WORKERS
CHUNK = 128
NCHUNK = B_PER_W // CHUNK
TOTAL = NLOOKUP * NCHUNK
NBUF = 4
DEPTH = NBUF - 1


def _body(idx_hbm, ent, rel, ts,
          out_h, out_r, out_t, out_ts,
          idx_v, rows_v, gsem, ssem):
    wid = lax.axis_index("s") * NUM_CORES + lax.axis_index("c")
    base = wid * B_PER_W

    pltpu.sync_copy(idx_hbm.at[wid], idx_v)

    tables = (ent, rel, ent, ts)
    outs = (out_h, out_r, out_t, out_ts)
    g = [None] * TOTAL
    s = [None] * TOTAL

    def start_gather(c):
        l, j = divmod(c, NCHUNK)
        buf = c % NBUF
        g[c] = pltpu.async_copy(
            tables[l].at[idx_v.at[l, j]], rows_v.at[buf], gsem.at[buf])

    def start_store(c):
        l, j = divmod(c, NCHUNK)
        buf = c % NBUF
        s[c] = pltpu.async_copy(
            rows_v.at[buf],
            outs[l].at[pl.ds(base + j * CHUNK, CHUNK)],
            ssem.at[buf])

    for c in range(TOTAL + DEPTH):
        if c < TOTAL:
            if c >= NBUF:
                s[c - NBUF].wait()
            start_gather(c)
        d = c - DEPTH
        if 0 <= d < TOTAL:
            g[d].wait()
            start_store(d)
    for d in range(TOTAL - NBUF, TOTAL):
        s[d].wait()


@jax.jit
def _gather4(idx, entity_table, relation_table, timestamp_table):
    mesh = plsc.VectorSubcoreMesh(core_axis_name="c", subcore_axis_name="s")
    out = jax.ShapeDtypeStruct((BATCH, DIM), jnp.float32)
    return pl.kernel(
        _body,
        out_type=(out, out, out, out),
        mesh=mesh,
        scratch_types=[
            pltpu.VMEM((NLOOKUP, NCHUNK, CHUNK), jnp.int32),
            pltpu.VMEM((NBUF, CHUNK, DIM), jnp.float32),
            pltpu.SemaphoreType.DMA((NBUF,)),
            pltpu.SemaphoreType.DMA((NBUF,)),
        ],
    )(idx, entity_table, relation_table, timestamp_table)


def kernel(head, relation, tail, timestamp,
           entity_table, relation_table, timestamp_table):
    idx = jnp.stack([
        head.astype(jnp.int32),
        relation.astype(jnp.int32),
        tail.astype(jnp.int32),
        timestamp.astype(jnp.int32),
    ])
    idx = idx.reshape(NLOOKUP, NUM_WORKERS, NCHUNK, CHUNK).transpose(1, 0, 2, 3)
    return _gather4(idx, entity_table, relation_table, timestamp_table)


# speedup vs baseline: 1.2178x; 1.2178x over previous
import functools

import jax
import jax.numpy as jnp
from jax import lax
from jax.experimental import pallas as pl
from jax.experimental.pallas import tpu as pltpu
from jax.experimental.pallas import tpu_sc as plsc

NUM_CORES = 2
NUM_SUBCORES = 16
NUM_WORKERS = NUM_CORES * NUM_SUBCORES

BATCH = 16384
DIM = 64
NLOOKUP = 4

B_PER_W = BATCH // NUM_WORKERS
CHUNK = 128
NCHUNK = B_PER_W // CHUNK
TOTAL = NLOOKUP * NCHUNK
NBUF = 4
DEPTH = NBUF - 1


def _body(idx_hbm, ent, rel, ts,
          out_h, out_r, out_t, out_ts,
          idx_v, rows_v, gsem, ssem):
    wid = lax.axis_index("s") * NUM_CORES + lax.axis_index("c")
    base = wid * B_PER_W

    pltpu.sync_copy(idx_hbm.at[wid], idx_v)

    tables = (ent, rel, ent, ts)
    outs = (out_h, out_r, out_t, out_ts)
    g = [None] * TOTAL
    s = [None] * TOTAL

    def start_gather(c):
        l, j = divmod(c, NCHUNK)
        buf = c % NBUF
        g[c] = pltpu.async_copy(
            tables[l].at[idx_v.at[l, j]], rows_v.at[buf], gsem.at[buf])

    def start_store(c):
        l, j = divmod(c, NCHUNK)
        buf = c % NBUF
        s[c] = pltpu.async_copy(
            rows_v.at[buf],
            outs[l].at[pl.ds(base + j * CHUNK, CHUNK)],
            ssem.at[buf])

    for c in range(TOTAL + DEPTH):
        if c < TOTAL:
            if c >= NBUF:
                s[c - NBUF].wait()
            start_gather(c)
        d = c - DEPTH
        if 0 <= d < TOTAL:
            g[d].wait()
            start_store(d)
    for d in range(TOTAL - NBUF, TOTAL):
        s[d].wait()


@jax.jit
def _gather4(idx, entity_table, relation_table, timestamp_table):
    mesh = plsc.VectorSubcoreMesh(core_axis_name="c", subcore_axis_name="s")
    out = jax.ShapeDtypeStruct((BATCH, DIM), jnp.float32)
    return pl.kernel(
        _body,
        out_type=(out, out, out, out),
        mesh=mesh,
        compiler_params=pltpu.CompilerParams(use_tc_tiling_on_sc=False),
        scratch_types=[
            pltpu.VMEM((NLOOKUP, NCHUNK, CHUNK), jnp.int32),
            pltpu.VMEM((NBUF, CHUNK, DIM), jnp.float32),
            pltpu.SemaphoreType.DMA((NBUF,)),
            pltpu.SemaphoreType.DMA((NBUF,)),
        ],
    )(idx, entity_table, relation_table, timestamp_table)


def kernel(head, relation, tail, timestamp,
           entity_table, relation_table, timestamp_table):
    idx = jnp.stack([
        head.astype(jnp.int32),
        relation.astype(jnp.int32),
        tail.astype(jnp.int32),
        timestamp.astype(jnp.int32),
    ])
    idx = idx.reshape(NLOOKUP, NUM_WORKERS, NCHUNK, CHUNK).transpose(1, 0, 2, 3)
    return _gather4(idx, entity_table, relation_table, timestamp_table)


# speedup vs baseline: 1.2353x; 1.0144x over previous
import functools

import jax
import jax.numpy as jnp
from jax import lax
from jax.experimental import pallas as pl
from jax.experimental.pallas import tpu as pltpu
from jax.experimental.pallas import tpu_sc as plsc

NUM_CORES = 2
NUM_SUBCORES = 16
NUM_WORKERS = NUM_CORES * NUM_SUBCORES

BATCH = 16384
DIM = 64
NLOOKUP = 4

B_PER_W = BATCH // NUM_WORKERS
CHUNK = 512
NCHUNK = B_PER_W // CHUNK
TOTAL = NLOOKUP * NCHUNK
NBUF = 3
DEPTH = NBUF - 1


def _body(idx_hbm, ent, rel, ts,
          out_h, out_r, out_t, out_ts,
          idx_v, rows_v, gsem, ssem):
    wid = lax.axis_index("s") * NUM_CORES + lax.axis_index("c")
    base = wid * B_PER_W

    pltpu.sync_copy(idx_hbm.at[wid], idx_v)

    tables = (ent, rel, ent, ts)
    outs = (out_h, out_r, out_t, out_ts)
    g = [None] * TOTAL
    s = [None] * TOTAL

    def start_gather(c):
        l, j = divmod(c, NCHUNK)
        buf = c % NBUF
        g[c] = pltpu.async_copy(
            tables[l].at[idx_v.at[l, j]], rows_v.at[buf], gsem.at[buf])

    def start_store(c):
        l, j = divmod(c, NCHUNK)
        buf = c % NBUF
        s[c] = pltpu.async_copy(
            rows_v.at[buf],
            outs[l].at[pl.ds(base + j * CHUNK, CHUNK)],
            ssem.at[buf])

    for c in range(TOTAL + DEPTH):
        if c < TOTAL:
            if c >= NBUF:
                s[c - NBUF].wait()
            start_gather(c)
        d = c - DEPTH
        if 0 <= d < TOTAL:
            g[d].wait()
            start_store(d)
    for d in range(TOTAL - NBUF, TOTAL):
        s[d].wait()


@jax.jit
def _gather4(idx, entity_table, relation_table, timestamp_table):
    mesh = plsc.VectorSubcoreMesh(core_axis_name="c", subcore_axis_name="s")
    out = jax.ShapeDtypeStruct((BATCH, DIM), jnp.float32)
    return pl.kernel(
        _body,
        out_type=(out, out, out, out),
        mesh=mesh,
        compiler_params=pltpu.CompilerParams(use_tc_tiling_on_sc=False),
        scratch_types=[
            pltpu.VMEM((NLOOKUP, NCHUNK, CHUNK), jnp.int32),
            pltpu.VMEM((NBUF, CHUNK, DIM), jnp.float32),
            pltpu.SemaphoreType.DMA((NBUF,)),
            pltpu.SemaphoreType.DMA((NBUF,)),
        ],
    )(idx, entity_table, relation_table, timestamp_table)


def kernel(head, relation, tail, timestamp,
           entity_table, relation_table, timestamp_table):
    idx = jnp.stack([
        head.astype(jnp.int32),
        relation.astype(jnp.int32),
        tail.astype(jnp.int32),
        timestamp.astype(jnp.int32),
    ])
    idx = idx.reshape(NLOOKUP, NUM_WORKERS, NCHUNK, CHUNK).transpose(1, 0, 2, 3)
    return _gather4(idx, entity_table, relation_table, timestamp_table)


# speedup vs baseline: 1.4309x; 1.1583x over previous
import functools

import jax
import jax.numpy as jnp
from jax import lax
from jax.experimental import pallas as pl
from jax.experimental.pallas import tpu as pltpu
from jax.experimental.pallas import tpu_sc as plsc

NUM_CORES = 2
NUM_SUBCORES = 16
NUM_WORKERS = NUM_CORES * NUM_SUBCORES

BATCH = 16384
DIM = 64
PADDIM = 128
NLOOKUP = 4

B_PER_W = BATCH // NUM_WORKERS
CHUNK = 512
NCHUNK = B_PER_W // CHUNK
TOTAL = NLOOKUP * NCHUNK
NBUF = 3
DEPTH = NBUF - 1


def _body(idx_hbm, ent, rel, ts,
          out_h, out_r, out_t, out_ts,
          idx_v, rows_v, gsem, ssem):
    wid = lax.axis_index("s") * NUM_CORES + lax.axis_index("c")
    base = wid * B_PER_W

    pltpu.sync_copy(idx_hbm.at[wid], idx_v)

    tables = (ent, rel, ent, ts)
    outs = (out_h, out_r, out_t, out_ts)
    g = [None] * TOTAL
    s = [None] * TOTAL

    def start_gather(c):
        l, j = divmod(c, NCHUNK)
        buf = c % NBUF
        g[c] = pltpu.async_copy(
            tables[l].at[idx_v.at[l, j]], rows_v.at[buf], gsem.at[buf])

    def start_store(c):
        l, j = divmod(c, NCHUNK)
        buf = c % NBUF
        s[c] = pltpu.async_copy(
            rows_v.at[buf],
            outs[l].at[pl.ds(base + j * CHUNK, CHUNK), pl.ds(0, DIM)],
            ssem.at[buf])

    for c in range(TOTAL + DEPTH):
        if c < TOTAL:
            if c >= NBUF:
                s[c - NBUF].wait()
            start_gather(c)
        d = c - DEPTH
        if 0 <= d < TOTAL:
            g[d].wait()
            start_store(d)
    for d in range(TOTAL - NBUF, TOTAL):
        s[d].wait()


@jax.jit
def _gather4(idx, entity_table, relation_table, timestamp_table):
    mesh = plsc.VectorSubcoreMesh(core_axis_name="c", subcore_axis_name="s")
    out = jax.ShapeDtypeStruct((BATCH, PADDIM), jnp.float32)
    return pl.kernel(
        _body,
        out_type=(out, out, out, out),
        mesh=mesh,
        compiler_params=pltpu.CompilerParams(use_tc_tiling_on_sc=False),
        scratch_types=[
            pltpu.VMEM((NLOOKUP, NCHUNK, CHUNK), jnp.int32),
            pltpu.VMEM((NBUF, CHUNK, DIM), jnp.float32),
            pltpu.SemaphoreType.DMA((NBUF,)),
            pltpu.SemaphoreType.DMA((NBUF,)),
        ],
    )(idx, entity_table, relation_table, timestamp_table)


def kernel(head, relation, tail, timestamp,
           entity_table, relation_table, timestamp_table):
    idx = jnp.stack([
        head.astype(jnp.int32),
        relation.astype(jnp.int32),
        tail.astype(jnp.int32),
        timestamp.astype(jnp.int32),
    ])
    idx = idx.reshape(NLOOKUP, NUM_WORKERS, NCHUNK, CHUNK).transpose(1, 0, 2, 3)
    outs = _gather4(idx, entity_table, relation_table, timestamp_table)
    return tuple(o[:, :DIM] for o in outs)


# speedup vs baseline: 1.5263x; 1.0667x over previous
import functools

import jax
import jax.numpy as jnp
from jax import lax
from jax.experimental import pallas as pl
from jax.experimental.pallas import tpu as pltpu
from jax.experimental.pallas import tpu_sc as plsc

NUM_CORES = 2
NUM_SUBCORES = 16
NUM_WORKERS = NUM_CORES * NUM_SUBCORES

BATCH = 16384
DIM = 64
PADDIM = 128

B_PER_W = BATCH // NUM_WORKERS


def _pair_body(i0, i1, t0, t1, o0, o1, idx_v, rows_v, gsem, ssem):
    wid = lax.axis_index("s") * NUM_CORES + lax.axis_index("c")
    base = wid * B_PER_W

    pltpu.sync_copy(i0.at[pl.ds(base, B_PER_W)], idx_v.at[0])
    pltpu.sync_copy(i1.at[pl.ds(base, B_PER_W)], idx_v.at[1])

    g0 = pltpu.async_copy(t0.at[idx_v.at[0]], rows_v.at[0], gsem.at[0])
    g1 = pltpu.async_copy(t1.at[idx_v.at[1]], rows_v.at[1], gsem.at[1])

    dst = pl.ds(base, B_PER_W), pl.ds(0, DIM)
    g0.wait()
    s0 = pltpu.async_copy(rows_v.at[0], o0.at[dst], ssem.at[0])
    g1.wait()
    s1 = pltpu.async_copy(rows_v.at[1], o1.at[dst], ssem.at[1])
    s0.wait()
    s1.wait()


@jax.jit
def _pair(i0, i1, t0, t1):
    mesh = plsc.VectorSubcoreMesh(core_axis_name="c", subcore_axis_name="s")
    out = jax.ShapeDtypeStruct((BATCH, PADDIM), jnp.float32)
    return pl.kernel(
        _pair_body,
        out_type=(out, out),
        mesh=mesh,
        compiler_params=pltpu.CompilerParams(use_tc_tiling_on_sc=False),
        scratch_types=[
            pltpu.VMEM((2, B_PER_W), jnp.int32),
            pltpu.VMEM((2, B_PER_W, DIM), jnp.float32),
            pltpu.SemaphoreType.DMA((2,)),
            pltpu.SemaphoreType.DMA((2,)),
        ],
    )(i0, i1, t0, t1)


def kernel(head, relation, tail, timestamp,
           entity_table, relation_table, timestamp_table):
    rel_o, ts_o = _pair(relation.astype(jnp.int32),
                        timestamp.astype(jnp.int32),
                        relation_table, timestamp_table)
    head_o, tail_o = _pair(head.astype(jnp.int32),
                           tail.astype(jnp.int32),
                           entity_table, entity_table)
    return (head_o[:, :DIM], rel_o[:, :DIM],
            tail_o[:, :DIM], ts_o[:, :DIM])
